# Initial kernel scaffold; baseline (speedup 1.0000x reference)
import jax
import jax.numpy as jnp
from jax.experimental import pallas as pl


def kernel(latent, condition, edge_index, Wz, bz, Wc, bc, Wo, bo):
    raise NotImplementedError("write your pallas kernel here")


# speedup vs baseline: 8.8166x; 8.8166x over previous
import functools

import jax
import jax.numpy as jnp
from jax import lax
from jax.experimental import pallas as pl
from jax.experimental.pallas import tpu as pltpu
from jax.experimental.pallas import tpu_sc as plsc

N = 10000
D = 128
H = D // 2
NPAD = 10240
ZROW = 10000
TRASH = 10200
E = 320000
EPAD = 327680
CHUNK = 128
NROWS_E = EPAD // CHUNK
NCA = NROWS_E // 16
NCB = NROWS_E // 32
RPT = NPAD // 16

_MESH = plsc.VectorSubcoreMesh(core_axis_name="c", subcore_axis_name="s")


def _edge_sweep(tbl_hbm, idxs, idxd, rows, acc, sem, nchunks):
    def body(j, carry):
        pltpu.async_copy(tbl_hbm.at[idxs.at[j]], rows, sem).wait()
        pltpu.sync_copy(rows, acc.at[idxd.at[j]], add=True)
        return carry
    lax.fori_loop(0, nchunks, body, 0)


@functools.partial(
    pl.kernel,
    out_type=jax.ShapeDtypeStruct((2, NPAD), jnp.float32),
    mesh=_MESH,
    scratch_types=[
        pltpu.VMEM((NCB, CHUNK), jnp.int32),
        pltpu.VMEM((CHUNK,), jnp.float32),
        pltpu.VMEM((RPT,), jnp.float32),
        pltpu.VMEM_SHARED((NPAD,), jnp.float32),
    ],
)
def _deg_kernel(dst_hbm, out_hbm, idxd, ones, zbuf, deg_sh):
    c = lax.axis_index("c")
    s = lax.axis_index("s")
    w = c * 16 + s

    def fill_ones(i, carry):
        ones[pl.ds(i * 16, 16)] = jnp.full((16,), 1.0, jnp.float32)
        return carry
    lax.fori_loop(0, CHUNK // 16, fill_ones, 0)

    def fill_zeros(i, carry):
        zbuf[pl.ds(i * 16, 16)] = jnp.zeros((16,), jnp.float32)
        return carry
    lax.fori_loop(0, RPT // 16, fill_zeros, 0)

    pltpu.sync_copy(zbuf, deg_sh.at[pl.ds(s * RPT, RPT)])
    pltpu.sync_copy(dst_hbm.at[pl.ds(w * NCB, NCB)], idxd)
    plsc.subcore_barrier()

    def body(j, carry):
        pltpu.sync_copy(ones, deg_sh.at[idxd.at[j]], add=True)
        return carry
    lax.fori_loop(0, NCB, body, 0)

    plsc.subcore_barrier()
    pltpu.sync_copy(deg_sh.at[pl.ds(s * RPT, RPT)],
                    out_hbm.at[c, pl.ds(s * RPT, RPT)])


@functools.partial(
    pl.kernel,
    out_type=jax.ShapeDtypeStruct((2, NPAD, H), jnp.float32),
    mesh=_MESH,
    scratch_types=[
        pltpu.VMEM((NCA, CHUNK), jnp.int32),
        pltpu.VMEM((NCA, CHUNK), jnp.int32),
        pltpu.VMEM((CHUNK, H), jnp.float32),
        pltpu.VMEM_SHARED((NPAD, H), jnp.float32),
        pltpu.SemaphoreType.DMA,
    ],
    compiler_params=pltpu.CompilerParams(use_tc_tiling_on_sc=False),
)
def _sweep_kernel(tbla_hbm, tblb_hbm, src_hbm, dst_hbm, zeros_hbm, out_hbm,
                  idxs, idxd, rows, acc, sem):
    c = lax.axis_index("c")
    s = lax.axis_index("s")
    rslc = pl.ds(s * RPT, RPT)
    pltpu.sync_copy(zeros_hbm.at[rslc], acc.at[rslc])
    pltpu.sync_copy(src_hbm.at[pl.ds(s * NCA, NCA)], idxs)
    pltpu.sync_copy(dst_hbm.at[pl.ds(s * NCA, NCA)], idxd)
    plsc.subcore_barrier()

    @pl.when(c == 0)
    def _():
        _edge_sweep(tbla_hbm, idxs, idxd, rows, acc, sem, NCA)

    @pl.when(c == 1)
    def _():
        _edge_sweep(tblb_hbm, idxs, idxd, rows, acc, sem, NCA)

    plsc.subcore_barrier()
    pltpu.sync_copy(acc.at[rslc], out_hbm.at[c, rslc])


BLK = 1024
GRID = NPAD // BLK
_CONTRACT = (((1,), (1,)), ((), ()))


def _tc1_body(lat_ref, cond_ref, wz_ref, wc_ref, d0_ref, d1_ref,
              hs1a_ref, hs1b_ref, hs2a_ref, hs2b_ref, dinv_ref):
    deg = d0_ref[...] + d1_ref[...] + 1.0
    dinv = lax.rsqrt(deg)
    h1 = lax.dot_general(lat_ref[...], wz_ref[...], _CONTRACT,
                         preferred_element_type=jnp.float32) * dinv
    h2 = lax.dot_general(cond_ref[...], wc_ref[...], _CONTRACT,
                         preferred_element_type=jnp.float32) * dinv
    hs1a_ref[...] = h1[:, :H]
    hs1b_ref[...] = h1[:, H:]
    hs2a_ref[...] = h2[:, :H]
    hs2b_ref[...] = h2[:, H:]
    dinv_ref[...] = dinv


_half_out = jax.ShapeDtypeStruct((NPAD, H), jnp.float32)
_half_spec = pl.BlockSpec((BLK, H), lambda i: (i, 0))
_full_spec = pl.BlockSpec((BLK, D), lambda i: (i, 0))
_col_spec = pl.BlockSpec((BLK, 1), lambda i: (i, 0))

_tc1 = pl.pallas_call(
    _tc1_body,
    grid=(GRID,),
    in_specs=[
        _full_spec,
        _full_spec,
        pl.BlockSpec((D, D), lambda i: (0, 0)),
        pl.BlockSpec((D, D), lambda i: (0, 0)),
        _col_spec,
        _col_spec,
    ],
    out_specs=[_half_spec, _half_spec, _half_spec, _half_spec, _col_spec],
    out_shape=[_half_out, _half_out, _half_out, _half_out,
               jax.ShapeDtypeStruct((NPAD, 1), jnp.float32)],
)


def _tc2_body(za_ref, zb_ref, ca_ref, cb_ref, hs1a_ref, hs1b_ref,
              hs2a_ref, hs2b_ref, dinv_ref, bz_ref, bc_ref, wo_ref,
              hs3a_ref, hs3b_ref):
    dinv = dinv_ref[...]
    accz = jnp.concatenate([za_ref[...] + hs1a_ref[...],
                            zb_ref[...] + hs1b_ref[...]], axis=1)
    accc = jnp.concatenate([ca_ref[...] + hs2a_ref[...],
                            cb_ref[...] + hs2b_ref[...]], axis=1)
    z2h = jnp.tanh(accz * dinv + bz_ref[...])
    c2h = jnp.tanh(accc * dinv + bc_ref[...])
    wo = wo_ref[...]
    h3 = (lax.dot_general(z2h, wo[:, :D], _CONTRACT,
                          preferred_element_type=jnp.float32)
          + lax.dot_general(c2h, wo[:, D:], _CONTRACT,
                            preferred_element_type=jnp.float32)) * dinv
    hs3a_ref[...] = h3[:, :H]
    hs3b_ref[...] = h3[:, H:]


_tc2 = pl.pallas_call(
    _tc2_body,
    grid=(GRID,),
    in_specs=[
        _half_spec, _half_spec, _half_spec, _half_spec,
        _half_spec, _half_spec, _half_spec, _half_spec,
        _col_spec,
        pl.BlockSpec((1, D), lambda i: (0, 0)),
        pl.BlockSpec((1, D), lambda i: (0, 0)),
        pl.BlockSpec((D, 2 * D), lambda i: (0, 0)),
    ],
    out_specs=[_half_spec, _half_spec],
    out_shape=[_half_out, _half_out],
)


def _tc3_body(oa_ref, ob_ref, hs3a_ref, hs3b_ref, dinv_ref, bo_ref, out_ref):
    dinv = dinv_ref[...]
    acc = jnp.concatenate([oa_ref[...] + hs3a_ref[...],
                           ob_ref[...] + hs3b_ref[...]], axis=1)
    out_ref[...] = acc * dinv + bo_ref[...]


_tc3 = pl.pallas_call(
    _tc3_body,
    grid=(GRID,),
    in_specs=[
        _half_spec, _half_spec, _half_spec, _half_spec,
        _col_spec,
        pl.BlockSpec((1, D), lambda i: (0, 0)),
    ],
    out_specs=_full_spec,
    out_shape=jax.ShapeDtypeStruct((NPAD, D), jnp.float32),
)


def kernel(latent, condition, edge_index, Wz, bz, Wc, bc, Wo, bo):
    ei = edge_index.astype(jnp.int32)
    src2d = jnp.concatenate(
        [ei[0], jnp.full((EPAD - E,), ZROW, jnp.int32)]).reshape(NROWS_E, CHUNK)
    dst2d = jnp.concatenate(
        [ei[1], jnp.full((EPAD - E,), TRASH, jnp.int32)]).reshape(NROWS_E, CHUNK)
    latp = jnp.pad(latent, ((0, NPAD - N), (0, 0)))
    condp = jnp.pad(condition, ((0, NPAD - N), (0, 0)))
    zeros_nh = jnp.zeros((NPAD, H), jnp.float32)

    degp = _deg_kernel(dst2d)
    d0 = degp[0].reshape(NPAD, 1)
    d1 = degp[1].reshape(NPAD, 1)

    hs1a, hs1b, hs2a, hs2b, dinv = _tc1(latp, condp, Wz, Wc, d0, d1)
    accZ = _sweep_kernel(hs1a, hs1b, src2d, dst2d, zeros_nh)
    accC = _sweep_kernel(hs2a, hs2b, src2d, dst2d, zeros_nh)
    hs3a, hs3b = _tc2(accZ[0], accZ[1], accC[0], accC[1],
                      hs1a, hs1b, hs2a, hs2b, dinv,
                      bz.reshape(1, D), bc.reshape(1, D), Wo)
    accO = _sweep_kernel(hs3a, hs3b, src2d, dst2d, zeros_nh)
    out = _tc3(accO[0], accO[1], hs3a, hs3b, dinv, bo.reshape(1, D))
    return out[:N]


# speedup vs baseline: 11.8406x; 1.3430x over previous
import functools

import jax
import jax.numpy as jnp
from jax import lax
from jax.experimental import pallas as pl
from jax.experimental.pallas import tpu as pltpu
from jax.experimental.pallas import tpu_sc as plsc

N = 10000
D = 128
H = D // 2
NPAD = 10240
ZROW = 10000
TRASH = 10200
E = 320000
EPAD = 327680
CHUNK = 128
NROWS_E = EPAD // CHUNK
NCA = NROWS_E // 16
NCB = NROWS_E // 32
RPT = NPAD // 16

_MESH = plsc.VectorSubcoreMesh(core_axis_name="c", subcore_axis_name="s")


NB = 4


def _edge_sweep(tbl_hbm, idxs, idxd, rows, acc, sems, nchunks):
    ngroups = nchunks // NB
    for b in range(NB):
        pltpu.async_copy(tbl_hbm.at[idxs.at[b]], rows.at[b], sems[b])

    def body(g, carry):
        for b in range(NB):
            j = g * NB + b
            pltpu.make_async_copy(tbl_hbm.at[idxs.at[j]], rows.at[b],
                                  sems[b]).wait()
            pltpu.sync_copy(rows.at[b], acc.at[idxd.at[j]], add=True)
            pltpu.async_copy(tbl_hbm.at[idxs.at[j + NB]], rows.at[b], sems[b])
        return carry
    lax.fori_loop(0, ngroups - 1, body, 0)

    for b in range(NB):
        j = (ngroups - 1) * NB + b
        pltpu.make_async_copy(tbl_hbm.at[idxs.at[j]], rows.at[b],
                              sems[b]).wait()
        pltpu.sync_copy(rows.at[b], acc.at[idxd.at[j]], add=True)


@functools.partial(
    pl.kernel,
    out_type=jax.ShapeDtypeStruct((2, NPAD), jnp.float32),
    mesh=_MESH,
    scratch_types=[
        pltpu.VMEM((NCB, CHUNK), jnp.int32),
        pltpu.VMEM((CHUNK,), jnp.float32),
        pltpu.VMEM((RPT,), jnp.float32),
        pltpu.VMEM_SHARED((NPAD,), jnp.float32),
    ],
)
def _deg_kernel(dst_hbm, out_hbm, idxd, ones, zbuf, deg_sh):
    c = lax.axis_index("c")
    s = lax.axis_index("s")
    w = c * 16 + s

    def fill_ones(i, carry):
        ones[pl.ds(i * 16, 16)] = jnp.full((16,), 1.0, jnp.float32)
        return carry
    lax.fori_loop(0, CHUNK // 16, fill_ones, 0)

    def fill_zeros(i, carry):
        zbuf[pl.ds(i * 16, 16)] = jnp.zeros((16,), jnp.float32)
        return carry
    lax.fori_loop(0, RPT // 16, fill_zeros, 0)

    pltpu.sync_copy(zbuf, deg_sh.at[pl.ds(s * RPT, RPT)])
    pltpu.sync_copy(dst_hbm.at[pl.ds(w * NCB, NCB)], idxd)
    plsc.subcore_barrier()

    def body(j, carry):
        pltpu.sync_copy(ones, deg_sh.at[idxd.at[j]], add=True)
        return carry
    lax.fori_loop(0, NCB, body, 0)

    plsc.subcore_barrier()
    pltpu.sync_copy(deg_sh.at[pl.ds(s * RPT, RPT)],
                    out_hbm.at[c, pl.ds(s * RPT, RPT)])


@functools.partial(
    pl.kernel,
    out_type=jax.ShapeDtypeStruct((2, NPAD, H), jnp.float32),
    mesh=_MESH,
    scratch_types=[
        pltpu.VMEM((NCA, CHUNK), jnp.int32),
        pltpu.VMEM((NCA, CHUNK), jnp.int32),
        pltpu.VMEM((NB, CHUNK, H), jnp.float32),
        pltpu.VMEM_SHARED((NPAD, H), jnp.float32),
        pltpu.SemaphoreType.DMA,
        pltpu.SemaphoreType.DMA,
        pltpu.SemaphoreType.DMA,
        pltpu.SemaphoreType.DMA,
    ],
    compiler_params=pltpu.CompilerParams(use_tc_tiling_on_sc=False),
)
def _sweep_kernel(tbla_hbm, tblb_hbm, src_hbm, dst_hbm, zeros_hbm, out_hbm,
                  idxs, idxd, rows, acc, sem0, sem1, sem2, sem3):
    sems = (sem0, sem1, sem2, sem3)
    c = lax.axis_index("c")
    s = lax.axis_index("s")
    rslc = pl.ds(s * RPT, RPT)
    pltpu.sync_copy(zeros_hbm.at[rslc], acc.at[rslc])
    pltpu.sync_copy(src_hbm.at[pl.ds(s * NCA, NCA)], idxs)
    pltpu.sync_copy(dst_hbm.at[pl.ds(s * NCA, NCA)], idxd)
    plsc.subcore_barrier()

    @pl.when(c == 0)
    def _():
        _edge_sweep(tbla_hbm, idxs, idxd, rows, acc, sems, NCA)

    @pl.when(c == 1)
    def _():
        _edge_sweep(tblb_hbm, idxs, idxd, rows, acc, sems, NCA)

    plsc.subcore_barrier()
    pltpu.sync_copy(acc.at[rslc], out_hbm.at[c, rslc])


BLK = 1024
GRID = NPAD // BLK
_CONTRACT = (((1,), (1,)), ((), ()))


def _tc1_body(lat_ref, cond_ref, wz_ref, wc_ref, d0_ref, d1_ref,
              hs1a_ref, hs1b_ref, hs2a_ref, hs2b_ref, dinv_ref):
    deg = d0_ref[...] + d1_ref[...] + 1.0
    dinv = lax.rsqrt(deg)
    h1 = lax.dot_general(lat_ref[...], wz_ref[...], _CONTRACT,
                         preferred_element_type=jnp.float32) * dinv
    h2 = lax.dot_general(cond_ref[...], wc_ref[...], _CONTRACT,
                         preferred_element_type=jnp.float32) * dinv
    hs1a_ref[...] = h1[:, :H]
    hs1b_ref[...] = h1[:, H:]
    hs2a_ref[...] = h2[:, :H]
    hs2b_ref[...] = h2[:, H:]
    dinv_ref[...] = dinv


_half_out = jax.ShapeDtypeStruct((NPAD, H), jnp.float32)
_half_spec = pl.BlockSpec((BLK, H), lambda i: (i, 0))
_full_spec = pl.BlockSpec((BLK, D), lambda i: (i, 0))
_col_spec = pl.BlockSpec((BLK, 1), lambda i: (i, 0))

_tc1 = pl.pallas_call(
    _tc1_body,
    grid=(GRID,),
    in_specs=[
        _full_spec,
        _full_spec,
        pl.BlockSpec((D, D), lambda i: (0, 0)),
        pl.BlockSpec((D, D), lambda i: (0, 0)),
        _col_spec,
        _col_spec,
    ],
    out_specs=[_half_spec, _half_spec, _half_spec, _half_spec, _col_spec],
    out_shape=[_half_out, _half_out, _half_out, _half_out,
               jax.ShapeDtypeStruct((NPAD, 1), jnp.float32)],
)


def _tc2_body(za_ref, zb_ref, ca_ref, cb_ref, hs1a_ref, hs1b_ref,
              hs2a_ref, hs2b_ref, dinv_ref, bz_ref, bc_ref, wo_ref,
              hs3a_ref, hs3b_ref):
    dinv = dinv_ref[...]
    accz = jnp.concatenate([za_ref[...] + hs1a_ref[...],
                            zb_ref[...] + hs1b_ref[...]], axis=1)
    accc = jnp.concatenate([ca_ref[...] + hs2a_ref[...],
                            cb_ref[...] + hs2b_ref[...]], axis=1)
    z2h = jnp.tanh(accz * dinv + bz_ref[...])
    c2h = jnp.tanh(accc * dinv + bc_ref[...])
    wo = wo_ref[...]
    h3 = (lax.dot_general(z2h, wo[:, :D], _CONTRACT,
                          preferred_element_type=jnp.float32)
          + lax.dot_general(c2h, wo[:, D:], _CONTRACT,
                            preferred_element_type=jnp.float32)) * dinv
    hs3a_ref[...] = h3[:, :H]
    hs3b_ref[...] = h3[:, H:]


_tc2 = pl.pallas_call(
    _tc2_body,
    grid=(GRID,),
    in_specs=[
        _half_spec, _half_spec, _half_spec, _half_spec,
        _half_spec, _half_spec, _half_spec, _half_spec,
        _col_spec,
        pl.BlockSpec((1, D), lambda i: (0, 0)),
        pl.BlockSpec((1, D), lambda i: (0, 0)),
        pl.BlockSpec((D, 2 * D), lambda i: (0, 0)),
    ],
    out_specs=[_half_spec, _half_spec],
    out_shape=[_half_out, _half_out],
)


def _tc3_body(oa_ref, ob_ref, hs3a_ref, hs3b_ref, dinv_ref, bo_ref, out_ref):
    dinv = dinv_ref[...]
    acc = jnp.concatenate([oa_ref[...] + hs3a_ref[...],
                           ob_ref[...] + hs3b_ref[...]], axis=1)
    out_ref[...] = acc * dinv + bo_ref[...]


_tc3 = pl.pallas_call(
    _tc3_body,
    grid=(GRID,),
    in_specs=[
        _half_spec, _half_spec, _half_spec, _half_spec,
        _col_spec,
        pl.BlockSpec((1, D), lambda i: (0, 0)),
    ],
    out_specs=_full_spec,
    out_shape=jax.ShapeDtypeStruct((NPAD, D), jnp.float32),
)


def kernel(latent, condition, edge_index, Wz, bz, Wc, bc, Wo, bo):
    ei = edge_index.astype(jnp.int32)
    src2d = jnp.concatenate(
        [ei[0], jnp.full((EPAD - E,), ZROW, jnp.int32)]).reshape(NROWS_E, CHUNK)
    dst2d = jnp.concatenate(
        [ei[1], jnp.full((EPAD - E,), TRASH, jnp.int32)]).reshape(NROWS_E, CHUNK)
    latp = jnp.pad(latent, ((0, NPAD - N), (0, 0)))
    condp = jnp.pad(condition, ((0, NPAD - N), (0, 0)))
    zeros_nh = jnp.zeros((NPAD, H), jnp.float32)

    degp = _deg_kernel(dst2d)
    d0 = degp[0].reshape(NPAD, 1)
    d1 = degp[1].reshape(NPAD, 1)

    hs1a, hs1b, hs2a, hs2b, dinv = _tc1(latp, condp, Wz, Wc, d0, d1)
    accZ = _sweep_kernel(hs1a, hs1b, src2d, dst2d, zeros_nh)
    accC = _sweep_kernel(hs2a, hs2b, src2d, dst2d, zeros_nh)
    hs3a, hs3b = _tc2(accZ[0], accZ[1], accC[0], accC[1],
                      hs1a, hs1b, hs2a, hs2b, dinv,
                      bz.reshape(1, D), bc.reshape(1, D), Wo)
    accO = _sweep_kernel(hs3a, hs3b, src2d, dst2d, zeros_nh)
    out = _tc3(accO[0], accO[1], hs3a, hs3b, dinv, bo.reshape(1, D))
    return out[:N]
